# Initial kernel scaffold; baseline (speedup 1.0000x reference)
import jax
import jax.numpy as jnp
from jax.experimental import pallas as pl


def kernel(x, edge_index, W1, a1s, a1d, b1, W2, a2s, a2d, b2, Wo, bo):
    raise NotImplementedError("write your pallas kernel here")


# speedup vs baseline: 28.2100x; 28.2100x over previous
import jax
import jax.numpy as jnp
from jax import lax
from jax.experimental import pallas as pl
from jax.experimental.pallas import tpu as pltpu
from jax.experimental.pallas import tpu_sc as plsc

N = 10000
NPAD = 10240
F_IN = 128
E_RAW = 330000
EPAD = 331776
B = 128
PREC = lax.Precision.HIGHEST
F32 = jnp.float32


def _k1_body(x_ref, w1_ref, a1_ref, h_ref, esed_ref):
    h = jnp.dot(x_ref[...], w1_ref[...], preferred_element_type=F32,
                precision=PREC)
    h_ref[...] = h
    esed_ref[...] = jnp.dot(h, a1_ref[...], preferred_element_type=F32,
                            precision=PREC)


def _k1(xp, W1, A1):
    bm = 1280
    return pl.pallas_call(
        _k1_body,
        grid=(NPAD // bm,),
        in_specs=[pl.BlockSpec((bm, F_IN), lambda i: (i, 0)),
                  pl.BlockSpec((F_IN, F_IN), lambda i: (0, 0)),
                  pl.BlockSpec((F_IN, 16), lambda i: (0, 0))],
        out_specs=[pl.BlockSpec((bm, F_IN), lambda i: (i, 0)),
                   pl.BlockSpec((bm, 16), lambda i: (i, 0))],
        out_shape=[jax.ShapeDtypeStruct((NPAD, F_IN), F32),
                   jax.ShapeDtypeStruct((NPAD, 16), F32)],
    )(xp, W1, A1)


def _k3_body(a_ref, b_ref, pq0_ref, pq1_ref, b1_ref, w2_ref, a2_ref,
             h2_ref, ee_ref):
    M = (jnp.dot(a_ref[...], pq0_ref[...], preferred_element_type=F32,
                 precision=PREC)
         + jnp.dot(b_ref[...], pq1_ref[...], preferred_element_type=F32,
                   precision=PREC))
    h1 = M[:, :128] / (M[:, 128:] + 1e-16) + b1_ref[...]
    h1 = jnp.where(h1 > 0, h1, jnp.exp(h1) - 1.0)
    h2p = jnp.dot(h1, w2_ref[...], preferred_element_type=F32, precision=PREC)
    h2_ref[...] = h2p
    ee_ref[...] = jnp.dot(h2p, a2_ref[...], preferred_element_type=F32,
                          precision=PREC)


def _k3(acc_a, acc_b, PQ0, PQ1, b1, W2, A2):
    bm = 1280
    return pl.pallas_call(
        _k3_body,
        grid=(NPAD // bm,),
        in_specs=[pl.BlockSpec((bm, 80), lambda i: (i, 0)),
                  pl.BlockSpec((bm, 80), lambda i: (i, 0)),
                  pl.BlockSpec((80, 256), lambda i: (0, 0)),
                  pl.BlockSpec((80, 256), lambda i: (0, 0)),
                  pl.BlockSpec((1, 128), lambda i: (0, 0)),
                  pl.BlockSpec((128, 16), lambda i: (0, 0)),
                  pl.BlockSpec((16, 2), lambda i: (0, 0))],
        out_specs=[pl.BlockSpec((bm, 16), lambda i: (i, 0)),
                   pl.BlockSpec((bm, 2), lambda i: (i, 0))],
        out_shape=[jax.ShapeDtypeStruct((NPAD, 16), F32),
                   jax.ShapeDtypeStruct((NPAD, 2), F32)],
    )(acc_a, acc_b, PQ0, PQ1, b1, W2, A2)


def _k5_body(a_ref, b_ref, b2_ref, wo_ref, bo_ref, out_ref):
    M = a_ref[...] + b_ref[...]
    h2 = M[:, :16] / (M[:, 16:17] + 1e-16) + b2_ref[...]
    h2 = jnp.where(h2 > 0, h2, jnp.exp(h2) - 1.0)
    logits = jnp.dot(h2, wo_ref[...], preferred_element_type=F32,
                     precision=PREC) + bo_ref[...]
    m = jnp.max(logits, axis=1, keepdims=True)
    lse = jnp.log(jnp.sum(jnp.exp(logits - m), axis=1, keepdims=True)) + m
    out_ref[...] = logits - lse


def _k5(acc_a, acc_b, b2, Wo, bo):
    bm = 1280
    return pl.pallas_call(
        _k5_body,
        grid=(NPAD // bm,),
        in_specs=[pl.BlockSpec((bm, 32), lambda i: (i, 0)),
                  pl.BlockSpec((bm, 32), lambda i: (i, 0)),
                  pl.BlockSpec((1, 16), lambda i: (0, 0)),
                  pl.BlockSpec((16, 16), lambda i: (0, 0)),
                  pl.BlockSpec((1, 16), lambda i: (0, 0))],
        out_specs=[pl.BlockSpec((bm, 16), lambda i: (i, 0))],
        out_shape=[jax.ShapeDtypeStruct((NPAD, 16), F32)],
    )(acc_a, acc_b, b2, Wo, bo)


def _sc1_body(h_hbm, ed_hbm, src_hbm, dst_hbm, out_hbm,
              src_v, srcg_v, dst_v, dstg_v, rows_v, edr_v, prod_v, zrow_v,
              acc, sem, sem2):
    c = lax.axis_index("c")
    s = lax.axis_index("s")
    z16 = jnp.zeros((16,), F32)
    iota = lax.iota(jnp.int32, 16)

    @pl.loop(0, 400)
    def _(i):
        zrow_v[i // 5, pl.ds((i % 5) * 16, 16)] = z16

    @pl.loop(0, 8)
    def _(k):
        pltpu.sync_copy(zrow_v, acc.at[pl.ds(s * 640 + k * 80, 80)])

    @pl.loop(0, B)
    def _(i):
        prod_v[i, pl.ds(64, 16)] = z16

    plsc.subcore_barrier()

    @pl.loop(0, 162)
    def _(g):
        base = (s * 162 + g) * B
        pltpu.sync_copy(src_hbm.at[pl.ds(base, B)], src_v)
        pltpu.sync_copy(dst_hbm.at[pl.ds(base, B)], dst_v)

        @pl.loop(0, 8)
        def _(k):
            sl = pl.ds(k * 16, 16)
            srcg_v[sl] = src_v[sl] + c * NPAD
            dstg_v[sl] = dst_v[sl] + c * NPAD

        cp1 = pltpu.async_copy(h_hbm.at[srcg_v], rows_v, sem)
        cp2 = pltpu.async_copy(ed_hbm.at[dstg_v], edr_v, sem2)
        cp1.wait()
        cp2.wait()

        for j in range(8):
            eidx = iota + (j * 16)
            for t in range(4):
                esv = plsc.load_gather(
                    rows_v, [eidx, jnp.full((16,), 64 + t, jnp.int32)])
                edv = plsc.load_gather(
                    edr_v, [eidx, jnp.full((16,), t, jnp.int32)])
                e = esv + edv
                w = jnp.exp(jnp.where(e >= 0, e, 0.2 * e))
                plsc.store_scatter(
                    prod_v, [eidx, jnp.full((16,), 64 + t, jnp.int32)], w)
                for ch in range(16):
                    col = jnp.full((16,), t * 16 + ch, jnp.int32)
                    hv = plsc.load_gather(rows_v, [eidx, col])
                    plsc.store_scatter(prod_v, [eidx, col], hv * w)

        pltpu.sync_copy(prod_v, acc.at[dst_v], add=True)

    plsc.subcore_barrier()
    pltpu.sync_copy(acc.at[pl.ds(s * 640, 640)],
                    out_hbm.at[pl.ds(c * NPAD + s * 640, 640)])


def _sc1(h_cat, ed16, src, dst):
    fn = pl.kernel(
        _sc1_body,
        out_type=jax.ShapeDtypeStruct((2 * NPAD, 80), F32),
        mesh=plsc.VectorSubcoreMesh(core_axis_name="c", subcore_axis_name="s"),
        compiler_params=pltpu.CompilerParams(
            use_tc_tiling_on_sc=False, needs_layout_passes=False),
        scratch_types=[
            pltpu.VMEM((B,), jnp.int32),
            pltpu.VMEM((B,), jnp.int32),
            pltpu.VMEM((B,), jnp.int32),
            pltpu.VMEM((B,), jnp.int32),
            pltpu.VMEM((B, 80), F32),
            pltpu.VMEM((B, 16), F32),
            pltpu.VMEM((B, 80), F32),
            pltpu.VMEM((80, 80), F32),
            pltpu.VMEM_SHARED((NPAD, 80), F32),
            pltpu.SemaphoreType.DMA,
            pltpu.SemaphoreType.DMA,
        ],
    )
    return fn(h_cat, ed16, src, dst)


def _sc2_body(h2_hbm, ee_hbm, src_hbm, dst_hbm, out_hbm,
              ee_v, src_v, dst_v, rows_v, prod_v, zrow_v, acc, sem):
    c = lax.axis_index("c")
    s = lax.axis_index("s")
    w_id = c * 16 + s
    z16 = jnp.zeros((16,), F32)
    iota = lax.iota(jnp.int32, 16)
    zz = jnp.zeros((16,), jnp.int32)
    oo = jnp.full((16,), 1, jnp.int32)

    pltpu.sync_copy(ee_hbm, ee_v)

    @pl.loop(0, 160)
    def _(i):
        zrow_v[i // 2, pl.ds((i % 2) * 16, 16)] = z16

    @pl.loop(0, 8)
    def _(k):
        pltpu.sync_copy(zrow_v, acc.at[pl.ds(s * 640 + k * 80, 80)])

    @pl.loop(0, B)
    def _(i):
        prod_v[i, pl.ds(16, 16)] = z16

    plsc.subcore_barrier()

    @pl.loop(0, 81)
    def _(g):
        base = (w_id * 81 + g) * B
        pltpu.sync_copy(src_hbm.at[pl.ds(base, B)], src_v)
        pltpu.sync_copy(dst_hbm.at[pl.ds(base, B)], dst_v)
        pltpu.async_copy(h2_hbm.at[src_v], rows_v, sem).wait()

        for j in range(8):
            sl = pl.ds(j * 16, 16)
            eidx = iota + (j * 16)
            srcx = src_v[sl]
            dstx = dst_v[sl]
            esv = plsc.load_gather(ee_v, [zz, srcx])
            edv = plsc.load_gather(ee_v, [oo, dstx])
            e = esv + edv
            w = jnp.exp(jnp.where(e >= 0, e, 0.2 * e))
            plsc.store_scatter(prod_v, [eidx, jnp.full((16,), 16, jnp.int32)],
                               w)
            for ch in range(16):
                col = jnp.full((16,), ch, jnp.int32)
                hv = plsc.load_gather(rows_v, [eidx, col])
                plsc.store_scatter(prod_v, [eidx, col], hv * w)

        pltpu.sync_copy(prod_v, acc.at[dst_v], add=True)

    plsc.subcore_barrier()
    pltpu.sync_copy(acc.at[pl.ds(s * 640, 640)],
                    out_hbm.at[pl.ds(c * NPAD + s * 640, 640)])


def _sc2(h2pre, eed2, src, dst):
    fn = pl.kernel(
        _sc2_body,
        out_type=jax.ShapeDtypeStruct((2 * NPAD, 32), F32),
        mesh=plsc.VectorSubcoreMesh(core_axis_name="c", subcore_axis_name="s"),
        compiler_params=pltpu.CompilerParams(
            use_tc_tiling_on_sc=False, needs_layout_passes=False),
        scratch_types=[
            pltpu.VMEM((2, NPAD), F32),
            pltpu.VMEM((B,), jnp.int32),
            pltpu.VMEM((B,), jnp.int32),
            pltpu.VMEM((B, 16), F32),
            pltpu.VMEM((B, 32), F32),
            pltpu.VMEM((80, 32), F32),
            pltpu.VMEM_SHARED((NPAD, 32), F32),
            pltpu.SemaphoreType.DMA,
        ],
    )
    return fn(h2pre, eed2, src, dst)


def kernel(x, edge_index, W1, a1s, a1d, b1, W2, a2s, a2d, b2, Wo, bo):
    xp = jnp.pad(x, ((0, NPAD - N), (0, 0)))
    loops = jnp.arange(N, dtype=jnp.int32)
    pad = EPAD - E_RAW
    src = jnp.concatenate(
        [edge_index[0].astype(jnp.int32), loops, jnp.zeros((pad,), jnp.int32)])
    dst = jnp.concatenate(
        [edge_index[1].astype(jnp.int32), loops, jnp.full((pad,), N, jnp.int32)])

    rows = jnp.arange(F_IN)
    head = rows // 16
    A1 = jnp.zeros((F_IN, 16), F32)
    A1 = A1.at[rows, head].set(a1s.reshape(-1))
    A1 = A1.at[rows, head + 8].set(a1d.reshape(-1))

    h, esed = _k1(xp, W1, A1)
    zpad = jnp.zeros((2, NPAD, 12), F32)
    h_cat = jnp.concatenate(
        [h.reshape(NPAD, 2, 64).transpose(1, 0, 2),
         esed[:, :8].reshape(NPAD, 2, 4).transpose(1, 0, 2),
         zpad], axis=2).reshape(2 * NPAD, 80)
    ed16 = jnp.concatenate(
        [esed[:, 8:].reshape(NPAD, 2, 4).transpose(1, 0, 2),
         zpad], axis=2).reshape(2 * NPAD, 16)

    acc1 = _sc1(h_cat, ed16, src, dst)

    j64 = jnp.arange(64)
    tt = jnp.repeat(jnp.arange(4), 16)
    cc = jnp.tile(jnp.arange(16), 4)
    PQ0 = jnp.zeros((80, 256), F32)
    PQ0 = PQ0.at[j64, j64].set(1.0)
    PQ0 = PQ0.at[64 + tt, 128 + tt * 16 + cc].set(1.0)
    PQ1 = jnp.zeros((80, 256), F32)
    PQ1 = PQ1.at[j64, 64 + j64].set(1.0)
    PQ1 = PQ1.at[64 + tt, 192 + tt * 16 + cc].set(1.0)
    A2 = jnp.stack([a2s.reshape(-1), a2d.reshape(-1)], axis=1)

    h2pre, esed2 = _k3(acc1[:NPAD], acc1[NPAD:], PQ0, PQ1,
                       b1.reshape(1, -1), W2, A2)

    acc2 = _sc2(h2pre, esed2.T, src, dst)

    (out,) = _k5(acc2[:NPAD], acc2[NPAD:], b2.reshape(1, -1), Wo,
                 bo.reshape(1, -1))
    return out[:N]


# speedup vs baseline: 35.5960x; 1.2618x over previous
import jax
import jax.numpy as jnp
from jax import lax
from jax.experimental import pallas as pl
from jax.experimental.pallas import tpu as pltpu
from jax.experimental.pallas import tpu_sc as plsc

N = 10000
NPAD = 10240
F_IN = 128
E_RAW = 330000
EPAD = 331776
B = 128
PREC = lax.Precision.HIGHEST
F32 = jnp.float32


def _k1_body(x_ref, w1_ref, a1_ref, h_ref, esed_ref):
    h = jnp.dot(x_ref[...], w1_ref[...], preferred_element_type=F32,
                precision=PREC)
    h_ref[...] = h
    esed_ref[...] = jnp.dot(h, a1_ref[...], preferred_element_type=F32,
                            precision=PREC)


def _k1(xp, W1, A1):
    bm = 1280
    return pl.pallas_call(
        _k1_body,
        grid=(NPAD // bm,),
        in_specs=[pl.BlockSpec((bm, F_IN), lambda i: (i, 0)),
                  pl.BlockSpec((F_IN, F_IN), lambda i: (0, 0)),
                  pl.BlockSpec((F_IN, 16), lambda i: (0, 0))],
        out_specs=[pl.BlockSpec((bm, F_IN), lambda i: (i, 0)),
                   pl.BlockSpec((bm, 16), lambda i: (i, 0))],
        out_shape=[jax.ShapeDtypeStruct((NPAD, F_IN), F32),
                   jax.ShapeDtypeStruct((NPAD, 16), F32)],
    )(xp, W1, A1)


def _k3_body(a_ref, b_ref, pq0_ref, pq1_ref, b1_ref, w2_ref, a2_ref,
             h2_ref, ee_ref):
    M = (jnp.dot(a_ref[...], pq0_ref[...], preferred_element_type=F32,
                 precision=PREC)
         + jnp.dot(b_ref[...], pq1_ref[...], preferred_element_type=F32,
                   precision=PREC))
    h1 = M[:, :128] / (M[:, 128:] + 1e-16) + b1_ref[...]
    h1 = jnp.where(h1 > 0, h1, jnp.exp(h1) - 1.0)
    h2p = jnp.dot(h1, w2_ref[...], preferred_element_type=F32, precision=PREC)
    h2_ref[...] = h2p
    ee_ref[...] = jnp.dot(h2p, a2_ref[...], preferred_element_type=F32,
                          precision=PREC)


def _k3(acc_a, acc_b, PQ0, PQ1, b1, W2, A2):
    bm = 1280
    return pl.pallas_call(
        _k3_body,
        grid=(NPAD // bm,),
        in_specs=[pl.BlockSpec((bm, 80), lambda i: (i, 0)),
                  pl.BlockSpec((bm, 80), lambda i: (i, 0)),
                  pl.BlockSpec((80, 256), lambda i: (0, 0)),
                  pl.BlockSpec((80, 256), lambda i: (0, 0)),
                  pl.BlockSpec((1, 128), lambda i: (0, 0)),
                  pl.BlockSpec((128, 16), lambda i: (0, 0)),
                  pl.BlockSpec((16, 2), lambda i: (0, 0))],
        out_specs=[pl.BlockSpec((bm, 16), lambda i: (i, 0)),
                   pl.BlockSpec((bm, 2), lambda i: (i, 0))],
        out_shape=[jax.ShapeDtypeStruct((NPAD, 16), F32),
                   jax.ShapeDtypeStruct((NPAD, 2), F32)],
    )(acc_a, acc_b, PQ0, PQ1, b1, W2, A2)


def _k5_body(a_ref, b_ref, b2_ref, wo_ref, bo_ref, out_ref):
    M = a_ref[...] + b_ref[...]
    h2 = M[:, :16] / (M[:, 16:17] + 1e-16) + b2_ref[...]
    h2 = jnp.where(h2 > 0, h2, jnp.exp(h2) - 1.0)
    logits = jnp.dot(h2, wo_ref[...], preferred_element_type=F32,
                     precision=PREC) + bo_ref[...]
    m = jnp.max(logits, axis=1, keepdims=True)
    lse = jnp.log(jnp.sum(jnp.exp(logits - m), axis=1, keepdims=True)) + m
    out_ref[...] = logits - lse


def _k5(acc_a, acc_b, b2, Wo, bo):
    bm = 1280
    return pl.pallas_call(
        _k5_body,
        grid=(NPAD // bm,),
        in_specs=[pl.BlockSpec((bm, 32), lambda i: (i, 0)),
                  pl.BlockSpec((bm, 32), lambda i: (i, 0)),
                  pl.BlockSpec((1, 16), lambda i: (0, 0)),
                  pl.BlockSpec((16, 16), lambda i: (0, 0)),
                  pl.BlockSpec((1, 16), lambda i: (0, 0))],
        out_specs=[pl.BlockSpec((bm, 16), lambda i: (i, 0))],
        out_shape=[jax.ShapeDtypeStruct((NPAD, 16), F32)],
    )(acc_a, acc_b, b2, Wo, bo)


NB = 3


def _sc1_compute(rows_v, edr_v, prod_v, iota):
    @pl.loop(0, 8)
    def _(j):
        eidx = iota + (j * 16)
        for t in range(4):
            esv = plsc.load_gather(
                rows_v, [eidx, jnp.full((16,), 64 + t, jnp.int32)])
            edv = plsc.load_gather(
                edr_v, [eidx, jnp.full((16,), t, jnp.int32)])
            e = esv + edv
            w = jnp.exp(jnp.where(e >= 0, e, 0.2 * e))
            plsc.store_scatter(
                prod_v, [eidx, jnp.full((16,), 64 + t, jnp.int32)], w)
            for ch in range(16):
                col = jnp.full((16,), t * 16 + ch, jnp.int32)
                hv = plsc.load_gather(rows_v, [eidx, col])
                plsc.store_scatter(prod_v, [eidx, col], hv * w)


def _sc1_body(h_hbm, ed_hbm, srcg_hbm, dstg_hbm, dst_hbm, out_hbm, *refs):
    c = lax.axis_index("c")
    s = lax.axis_index("s")
    z16 = jnp.zeros((16,), F32)
    iota = lax.iota(jnp.int32, 16)

    srcg_b = refs[0:NB]
    dstg_b = refs[NB:2 * NB]
    dst_b = refs[2 * NB:3 * NB]
    rows_b = refs[3 * NB:4 * NB]
    edr_b = refs[4 * NB:5 * NB]
    prod_b = refs[5 * NB:6 * NB]
    zrow_v = refs[6 * NB]
    acc = refs[6 * NB + 1]
    semC = refs[6 * NB + 2:6 * NB + 2 + NB]
    semG = refs[6 * NB + 2 + NB:6 * NB + 2 + 2 * NB]
    semS = refs[6 * NB + 2 + 2 * NB:6 * NB + 2 + 3 * NB]

    @pl.loop(0, 200)
    def _(i):
        zrow_v[i // 5, pl.ds((i % 5) * 16, 16)] = z16

    @pl.loop(0, 16)
    def _(k):
        pltpu.sync_copy(zrow_v, acc.at[pl.ds(s * 640 + k * 40, 40)])

    for b in range(NB):
        @pl.loop(0, B)
        def _(i, _p=prod_b[b]):
            _p[i, pl.ds(64, 16)] = z16

    plsc.subcore_barrier()

    @pl.loop(0, 162 // NB)
    def _(p):
        g0 = (s * 162 + p * NB) * B
        idx_cp = []
        for b in range(NB):
            cpi = pltpu.async_copy(
                srcg_hbm.at[pl.ds(c * EPAD + g0 + b * B, B)],
                srcg_b[b], semC[b])
            cpj = pltpu.async_copy(
                dstg_hbm.at[pl.ds(c * EPAD + g0 + b * B, B)],
                dstg_b[b], semC[b])
            cpk = pltpu.async_copy(
                dst_hbm.at[pl.ds(g0 + b * B, B)], dst_b[b], semC[b])
            idx_cp.append((cpi, cpj, cpk))
        g_cp = []
        for b in range(NB):
            for cp in idx_cp[b]:
                cp.wait()
            cpa = pltpu.async_copy(h_hbm.at[srcg_b[b]], rows_b[b], semG[b])
            cpb = pltpu.async_copy(ed_hbm.at[dstg_b[b]], edr_b[b], semG[b])
            g_cp.append((cpa, cpb))
        s_cp = []
        for b in range(NB):
            for cp in g_cp[b]:
                cp.wait()
            _sc1_compute(rows_b[b], edr_b[b], prod_b[b], iota)
            s_cp.append(pltpu.async_copy(
                prod_b[b], acc.at[dst_b[b]], semS[b], add=True))
        for cp in s_cp:
            cp.wait()

    plsc.subcore_barrier()
    pltpu.sync_copy(acc.at[pl.ds(s * 640, 640)],
                    out_hbm.at[pl.ds(c * NPAD + s * 640, 640)])


def _sc1(h_cat, ed16, srcg2, dstg2, dst):
    fn = pl.kernel(
        _sc1_body,
        out_type=jax.ShapeDtypeStruct((2 * NPAD, 80), F32),
        mesh=plsc.VectorSubcoreMesh(core_axis_name="c", subcore_axis_name="s"),
        compiler_params=pltpu.CompilerParams(
            use_tc_tiling_on_sc=False, needs_layout_passes=False),
        scratch_types=(
            [pltpu.VMEM((B,), jnp.int32)] * NB
            + [pltpu.VMEM((B,), jnp.int32)] * NB
            + [pltpu.VMEM((B,), jnp.int32)] * NB
            + [pltpu.VMEM((B, 80), F32)] * NB
            + [pltpu.VMEM((B, 16), F32)] * NB
            + [pltpu.VMEM((B, 80), F32)] * NB
            + [pltpu.VMEM((40, 80), F32)]
            + [pltpu.VMEM_SHARED((NPAD, 80), F32)]
            + [pltpu.SemaphoreType.DMA] * (3 * NB)
        ),
    )
    return fn(h_cat, ed16, srcg2, dstg2, dst)


def _sc2_compute(ee_v, src_v, dst_v, rows_v, prod_v, iota, zz, oo):
    @pl.loop(0, 8)
    def _(j):
        sl = pl.ds(j * 16, 16)
        eidx = iota + (j * 16)
        srcx = src_v[sl]
        dstx = dst_v[sl]
        esv = plsc.load_gather(ee_v, [zz, srcx])
        edv = plsc.load_gather(ee_v, [oo, dstx])
        e = esv + edv
        w = jnp.exp(jnp.where(e >= 0, e, 0.2 * e))
        plsc.store_scatter(prod_v, [eidx, jnp.full((16,), 16, jnp.int32)], w)
        for ch in range(16):
            col = jnp.full((16,), ch, jnp.int32)
            hv = plsc.load_gather(rows_v, [eidx, col])
            plsc.store_scatter(prod_v, [eidx, col], hv * w)


def _sc2_body(h2_hbm, ee_hbm, src_hbm, dst_hbm, out_hbm, *refs):
    c = lax.axis_index("c")
    s = lax.axis_index("s")
    w_id = c * 16 + s
    z16 = jnp.zeros((16,), F32)
    iota = lax.iota(jnp.int32, 16)
    zz = jnp.zeros((16,), jnp.int32)
    oo = jnp.full((16,), 1, jnp.int32)

    ee_v = refs[0]
    src_b = refs[1:1 + NB]
    dst_b = refs[1 + NB:1 + 2 * NB]
    rows_b = refs[1 + 2 * NB:1 + 3 * NB]
    prod_b = refs[1 + 3 * NB:1 + 4 * NB]
    zrow_v = refs[1 + 4 * NB]
    acc = refs[2 + 4 * NB]
    semC = refs[3 + 4 * NB:3 + 5 * NB]
    semG = refs[3 + 5 * NB:3 + 6 * NB]
    semS = refs[3 + 6 * NB:3 + 7 * NB]

    pltpu.sync_copy(ee_hbm, ee_v)

    @pl.loop(0, 80)
    def _(i):
        zrow_v[i // 2, pl.ds((i % 2) * 16, 16)] = z16

    @pl.loop(0, 16)
    def _(k):
        pltpu.sync_copy(zrow_v, acc.at[pl.ds(s * 640 + k * 40, 40)])

    for b in range(NB):
        @pl.loop(0, B)
        def _(i, _p=prod_b[b]):
            _p[i, pl.ds(16, 16)] = z16

    plsc.subcore_barrier()

    @pl.loop(0, 81 // NB)
    def _(p):
        g0 = (w_id * 81 + p * NB) * B
        idx_cp = []
        for b in range(NB):
            cpi = pltpu.async_copy(
                src_hbm.at[pl.ds(g0 + b * B, B)], src_b[b], semC[b])
            cpj = pltpu.async_copy(
                dst_hbm.at[pl.ds(g0 + b * B, B)], dst_b[b], semC[b])
            idx_cp.append((cpi, cpj))
        g_cp = []
        for b in range(NB):
            for cp in idx_cp[b]:
                cp.wait()
            g_cp.append(pltpu.async_copy(
                h2_hbm.at[src_b[b]], rows_b[b], semG[b]))
        s_cp = []
        for b in range(NB):
            g_cp[b].wait()
            _sc2_compute(ee_v, src_b[b], dst_b[b], rows_b[b], prod_b[b],
                         iota, zz, oo)
            s_cp.append(pltpu.async_copy(
                prod_b[b], acc.at[dst_b[b]], semS[b], add=True))
        for cp in s_cp:
            cp.wait()

    plsc.subcore_barrier()
    pltpu.sync_copy(acc.at[pl.ds(s * 640, 640)],
                    out_hbm.at[pl.ds(c * NPAD + s * 640, 640)])


def _sc2(h2pre, eed2, src, dst):
    fn = pl.kernel(
        _sc2_body,
        out_type=jax.ShapeDtypeStruct((2 * NPAD, 32), F32),
        mesh=plsc.VectorSubcoreMesh(core_axis_name="c", subcore_axis_name="s"),
        compiler_params=pltpu.CompilerParams(
            use_tc_tiling_on_sc=False, needs_layout_passes=False),
        scratch_types=(
            [pltpu.VMEM((2, NPAD), F32)]
            + [pltpu.VMEM((B,), jnp.int32)] * NB
            + [pltpu.VMEM((B,), jnp.int32)] * NB
            + [pltpu.VMEM((B, 16), F32)] * NB
            + [pltpu.VMEM((B, 32), F32)] * NB
            + [pltpu.VMEM((40, 32), F32)]
            + [pltpu.VMEM_SHARED((NPAD, 32), F32)]
            + [pltpu.SemaphoreType.DMA] * (3 * NB)
        ),
    )
    return fn(h2pre, eed2, src, dst)


def kernel(x, edge_index, W1, a1s, a1d, b1, W2, a2s, a2d, b2, Wo, bo):
    xp = jnp.pad(x, ((0, NPAD - N), (0, 0)))
    loops = jnp.arange(N, dtype=jnp.int32)
    pad = EPAD - E_RAW
    src = jnp.concatenate(
        [edge_index[0].astype(jnp.int32), loops, jnp.zeros((pad,), jnp.int32)])
    dst = jnp.concatenate(
        [edge_index[1].astype(jnp.int32), loops, jnp.full((pad,), N, jnp.int32)])

    rows = jnp.arange(F_IN)
    head = rows // 16
    A1 = jnp.zeros((F_IN, 16), F32)
    A1 = A1.at[rows, head].set(a1s.reshape(-1))
    A1 = A1.at[rows, head + 8].set(a1d.reshape(-1))

    h, esed = _k1(xp, W1, A1)
    zpad = jnp.zeros((2, NPAD, 12), F32)
    h_cat = jnp.concatenate(
        [h.reshape(NPAD, 2, 64).transpose(1, 0, 2),
         esed[:, :8].reshape(NPAD, 2, 4).transpose(1, 0, 2),
         zpad], axis=2).reshape(2 * NPAD, 80)
    ed16 = jnp.concatenate(
        [esed[:, 8:].reshape(NPAD, 2, 4).transpose(1, 0, 2),
         zpad], axis=2).reshape(2 * NPAD, 16)

    srcg2 = jnp.concatenate([src, src + NPAD])
    dstg2 = jnp.concatenate([dst, dst + NPAD])
    acc1 = _sc1(h_cat, ed16, srcg2, dstg2, dst)

    j64 = jnp.arange(64)
    tt = jnp.repeat(jnp.arange(4), 16)
    cc = jnp.tile(jnp.arange(16), 4)
    PQ0 = jnp.zeros((80, 256), F32)
    PQ0 = PQ0.at[j64, j64].set(1.0)
    PQ0 = PQ0.at[64 + tt, 128 + tt * 16 + cc].set(1.0)
    PQ1 = jnp.zeros((80, 256), F32)
    PQ1 = PQ1.at[j64, 64 + j64].set(1.0)
    PQ1 = PQ1.at[64 + tt, 192 + tt * 16 + cc].set(1.0)
    A2 = jnp.stack([a2s.reshape(-1), a2d.reshape(-1)], axis=1)

    h2pre, esed2 = _k3(acc1[:NPAD], acc1[NPAD:], PQ0, PQ1,
                       b1.reshape(1, -1), W2, A2)

    acc2 = _sc2(h2pre, esed2.T, src, dst)

    (out,) = _k5(acc2[:NPAD], acc2[NPAD:], b2.reshape(1, -1), Wo,
                 bo.reshape(1, -1))
    return out[:N]
